# Initial kernel scaffold; baseline (speedup 1.0000x reference)
import jax
import jax.numpy as jnp
from jax.experimental import pallas as pl


def kernel(unknown, known, unknow_feats, known_feats, vdfeatures, lab_w1, lab_w2, conv0_w, bn0_gamma, bn0_beta, conv1_w, bn1_gamma, bn1_beta):
    raise NotImplementedError("write your pallas kernel here")


# speedup vs baseline: 16.0940x; 16.0940x over previous
import functools

import jax
import jax.numpy as jnp
from jax.experimental import pallas as pl

B, N, M = 8, 4096, 1024
C1, C2, VD, NVD = 64, 128, 32, 512
CIN, CMID, COUT = 224, 128, 128

NT = 512
GT = N // NT
NTOT = B * N

_HI = jax.lax.Precision.HIGHEST


def _gate_kernel(uf_ref, w1_ref, w2_ref, gate_ref):
    g = jnp.max(uf_ref[0], axis=1, keepdims=True)
    h = jnp.maximum(jnp.dot(w1_ref[...], g, precision=_HI), 0.0)
    z = jnp.dot(w2_ref[...], h, precision=_HI)
    gate_ref[0] = jax.nn.sigmoid(z)


def _main_kernel(u_ref, k_ref, kf_ref, uf_ref, vd_ref, gate_ref, w0_ref,
                 x0_ref, s0_ref, ss0_ref):
    b = pl.program_id(0)
    t = pl.program_id(1)

    u = u_ref[0]
    k = k_ref[0]
    u2 = jnp.sum(u * u, axis=0, keepdims=True)
    k2 = jnp.sum(k * k, axis=1, keepdims=True)
    ub = u.astype(jnp.bfloat16).astype(jnp.float32)
    kb = k.astype(jnp.bfloat16).astype(jnp.float32)
    acc = (kb[:, 0:1] * ub[0:1, :] + kb[:, 1:2] * ub[1:2, :]) + kb[:, 2:3] * ub[2:3, :]
    d2 = (u2 + k2) - 2.0 * acc

    iota_m = jax.lax.broadcasted_iota(jnp.int32, (M, 1), 0)
    inf = jnp.float32(jnp.inf)
    d2w = d2
    wt = jnp.zeros_like(d2)
    rs = []
    sels = []
    for _ in range(3):
        vj = jnp.min(d2w, axis=0, keepdims=True)
        ij = jnp.min(jnp.where(d2w == vj, iota_m, M), axis=0, keepdims=True)
        sel = iota_m == ij
        sels.append(sel)
        d2w = jnp.where(sel, inf, d2w)
        dj = jnp.sqrt(jnp.maximum(vj, 0.0))
        rs.append(1.0 / (dj + 1e-8))
    norm = (rs[0] + rs[1]) + rs[2]
    for sel, r in zip(sels, rs):
        wt = wt + jnp.where(sel, r / norm, 0.0)

    interp = jnp.dot(kf_ref[0], wt, precision=_HI)

    uf = uf_ref[0] * gate_ref[0]
    f = jnp.concatenate([interp, uf, vd_ref[0]], axis=0)
    x0 = jnp.dot(w0_ref[...], f, precision=_HI)
    x0_ref[0] = x0

    @pl.when(jnp.logical_and(b == 0, t == 0))
    def _():
        s0_ref[...] = jnp.zeros_like(s0_ref)
        ss0_ref[...] = jnp.zeros_like(ss0_ref)

    s0_ref[...] += jnp.sum(x0, axis=1, keepdims=True)
    ss0_ref[...] += jnp.sum(x0 * x0, axis=1, keepdims=True)


def _mid_kernel(x0_ref, s0_ref, ss0_ref, g0_ref, b0_ref, w1_ref,
                x1_ref, s1_ref, ss1_ref):
    b = pl.program_id(0)
    t = pl.program_id(1)
    mean = s0_ref[...] / NTOT
    var = ss0_ref[...] / NTOT - mean * mean
    rstd = jax.lax.rsqrt(var + 1e-5)
    scale = g0_ref[...] * rstd
    shift = b0_ref[...] - mean * scale
    h = jnp.maximum(x0_ref[0] * scale + shift, 0.0)
    x1 = jnp.dot(w1_ref[...], h, precision=_HI)
    x1_ref[0] = x1

    @pl.when(jnp.logical_and(b == 0, t == 0))
    def _():
        s1_ref[...] = jnp.zeros_like(s1_ref)
        ss1_ref[...] = jnp.zeros_like(ss1_ref)

    s1_ref[...] += jnp.sum(x1, axis=1, keepdims=True)
    ss1_ref[...] += jnp.sum(x1 * x1, axis=1, keepdims=True)


def _out_kernel(x1_ref, s1_ref, ss1_ref, g1_ref, b1_ref, out_ref):
    mean = s1_ref[...] / NTOT
    var = ss1_ref[...] / NTOT - mean * mean
    rstd = jax.lax.rsqrt(var + 1e-5)
    scale = g1_ref[...] * rstd
    shift = b1_ref[...] - mean * scale
    out_ref[0] = jnp.maximum(x1_ref[0] * scale + shift, 0.0)


def kernel(unknown, known, unknow_feats, known_feats, vdfeatures,
           lab_w1, lab_w2, conv0_w, bn0_gamma, bn0_beta,
           conv1_w, bn1_gamma, bn1_beta):
    f32 = jnp.float32
    uT = jnp.transpose(unknown, (0, 2, 1))

    gate = pl.pallas_call(
        _gate_kernel,
        grid=(B,),
        in_specs=[
            pl.BlockSpec((1, C1, N), lambda b: (b, 0, 0)),
            pl.BlockSpec((C1 // 4, C1), lambda b: (0, 0)),
            pl.BlockSpec((C1, C1 // 4), lambda b: (0, 0)),
        ],
        out_specs=pl.BlockSpec((1, C1, 1), lambda b: (b, 0, 0)),
        out_shape=jax.ShapeDtypeStruct((B, C1, 1), f32),
    )(unknow_feats, lab_w1, lab_w2)

    x0, s0, ss0 = pl.pallas_call(
        _main_kernel,
        grid=(B, GT),
        in_specs=[
            pl.BlockSpec((1, 3, NT), lambda b, t: (b, 0, t)),
            pl.BlockSpec((1, M, 3), lambda b, t: (b, 0, 0)),
            pl.BlockSpec((1, C2, M), lambda b, t: (b, 0, 0)),
            pl.BlockSpec((1, C1, NT), lambda b, t: (b, 0, t)),
            pl.BlockSpec((1, VD, NVD), lambda b, t: (b, 0, 0)),
            pl.BlockSpec((1, C1, 1), lambda b, t: (b, 0, 0)),
            pl.BlockSpec((CMID, CIN), lambda b, t: (0, 0)),
        ],
        out_specs=[
            pl.BlockSpec((1, CMID, NT), lambda b, t: (b, 0, t)),
            pl.BlockSpec((CMID, 1), lambda b, t: (0, 0)),
            pl.BlockSpec((CMID, 1), lambda b, t: (0, 0)),
        ],
        out_shape=[
            jax.ShapeDtypeStruct((B, CMID, N), f32),
            jax.ShapeDtypeStruct((CMID, 1), f32),
            jax.ShapeDtypeStruct((CMID, 1), f32),
        ],
    )(uT, known, known_feats, unknow_feats, vdfeatures, gate, conv0_w)

    x1, s1, ss1 = pl.pallas_call(
        _mid_kernel,
        grid=(B, GT),
        in_specs=[
            pl.BlockSpec((1, CMID, NT), lambda b, t: (b, 0, t)),
            pl.BlockSpec((CMID, 1), lambda b, t: (0, 0)),
            pl.BlockSpec((CMID, 1), lambda b, t: (0, 0)),
            pl.BlockSpec((CMID, 1), lambda b, t: (0, 0)),
            pl.BlockSpec((CMID, 1), lambda b, t: (0, 0)),
            pl.BlockSpec((COUT, CMID), lambda b, t: (0, 0)),
        ],
        out_specs=[
            pl.BlockSpec((1, COUT, NT), lambda b, t: (b, 0, t)),
            pl.BlockSpec((COUT, 1), lambda b, t: (0, 0)),
            pl.BlockSpec((COUT, 1), lambda b, t: (0, 0)),
        ],
        out_shape=[
            jax.ShapeDtypeStruct((B, COUT, N), f32),
            jax.ShapeDtypeStruct((COUT, 1), f32),
            jax.ShapeDtypeStruct((COUT, 1), f32),
        ],
    )(x0, s0, ss0, bn0_gamma.reshape(CMID, 1), bn0_beta.reshape(CMID, 1),
      conv1_w)

    out = pl.pallas_call(
        _out_kernel,
        grid=(B, GT),
        in_specs=[
            pl.BlockSpec((1, COUT, NT), lambda b, t: (b, 0, t)),
            pl.BlockSpec((COUT, 1), lambda b, t: (0, 0)),
            pl.BlockSpec((COUT, 1), lambda b, t: (0, 0)),
            pl.BlockSpec((COUT, 1), lambda b, t: (0, 0)),
            pl.BlockSpec((COUT, 1), lambda b, t: (0, 0)),
        ],
        out_specs=pl.BlockSpec((1, COUT, NT), lambda b, t: (b, 0, t)),
        out_shape=jax.ShapeDtypeStruct((B, COUT, N), f32),
    )(x1, s1, ss1, bn1_gamma.reshape(COUT, 1), bn1_beta.reshape(COUT, 1))

    return out


# speedup vs baseline: 23.3919x; 1.4535x over previous
import functools

import jax
import jax.numpy as jnp
from jax import lax
from jax.experimental import pallas as pl
from jax.experimental.pallas import tpu as pltpu
from jax.experimental.pallas import tpu_sc as plsc

B, N, M = 8, 4096, 1024
C1, C2, VD, NVD = 64, 128, 32, 512
CIN, CMID, COUT = 224, 128, 128

NT = 1024
GT = N // NT
NTOT = B * N

NC, NS = 2, 16
NW = NC * NS
QW = B * N // NW
CH = 128
NCH = QW // CH

_HI = jax.lax.Precision.HIGHEST


def _argmin_tree(x):
    idx = jax.lax.broadcasted_iota(jnp.int32, x.shape, 0)
    m = x.shape[0]
    n = x.shape[-1]
    while m > 1:
        h = m // 2
        x = x.reshape(2, h, n)
        idx = idx.reshape(2, h, n)
        cmp = x[0] <= x[1]
        x = jnp.where(cmp, x[0], x[1])
        idx = jnp.where(cmp, idx[0], idx[1])
        m = h
    return x, idx


def _gate_kernel(uf_ref, w1_ref, w2_ref, gate_ref):
    g = jnp.max(uf_ref[0], axis=1, keepdims=True)
    h = jnp.maximum(jnp.dot(w1_ref[...], g, precision=_HI), 0.0)
    z = jnp.dot(w2_ref[...], h, precision=_HI)
    gate_ref[0] = jax.nn.sigmoid(z)


def _nn_kernel(u_ref, k_ref, idx_ref, w_ref):
    b = pl.program_id(0)

    u = u_ref[0]
    k = k_ref[0]
    u2 = jnp.sum(u * u, axis=0, keepdims=True)
    k2 = jnp.sum(k * k, axis=1, keepdims=True)
    acc = jnp.dot(k, u)
    d2 = (u2 + k2) - 2.0 * acc

    iota_m = jax.lax.broadcasted_iota(jnp.int32, (M, 1), 0)
    inf = jnp.float32(jnp.inf)
    d2w = d2
    rs = []
    ijs = []
    for j in range(3):
        vj, ij = _argmin_tree(d2w)
        ijs.append(ij)
        dj = jnp.sqrt(jnp.maximum(vj, 0.0))
        rs.append(1.0 / (dj + 1e-8))
        if j < 2:
            sel = iota_m == ij
            d2w = jnp.where(sel, inf, d2w)
    norm = (rs[0] + rs[1]) + rs[2]
    idx_ref[0] = jnp.concatenate(ijs, axis=0) + b * M
    w_ref[0] = jnp.transpose(
        jnp.concatenate([r / norm for r in rs], axis=0), (1, 0))


def _sc_gather_body(table, idxg, out, i0_v, i1_v, i2_v,
                    r0_v, r1_v, r2_v, sem):
    wid = lax.axis_index("s") * NC + lax.axis_index("c")
    wpb = N // QW
    b = wid // wpb
    qoff = (wid % wpb) * QW
    for j, iv in enumerate((i0_v, i1_v, i2_v)):
        pltpu.sync_copy(idxg.at[pl.ds((b * 3 + j) * N + qoff, QW)], iv)
    for c in range(NCH):
        cps = [pltpu.async_copy(table.at[iv.at[pl.ds(c * CH, CH)]], rv, sem)
               for iv, rv in ((i0_v, r0_v), (i1_v, r1_v), (i2_v, r2_v))]
        for cp in cps:
            cp.wait()
        for j, rv in enumerate((r0_v, r1_v, r2_v)):
            dst = (j * B * N) + (b * N) + qoff + c * CH
            pltpu.sync_copy(rv, out.at[pl.ds(dst, CH), :])


def _conv0_kernel(g0_ref, g1_ref, g2_ref, wt_ref, uf_ref, vd_ref, gate_ref,
                  w0_ref, x0_ref, s0_ref, ss0_ref):
    b = pl.program_id(0)
    t = pl.program_id(1)
    w = wt_ref[0]
    interp_t = (w[:, 0:1] * g0_ref[...] + w[:, 1:2] * g1_ref[...]
                + w[:, 2:3] * g2_ref[...])
    interp = jnp.transpose(interp_t, (1, 0))
    uf = uf_ref[0] * gate_ref[0]
    vd = jnp.concatenate([vd_ref[0]] * (NT // NVD), axis=1)
    f = jnp.concatenate([interp, uf, vd], axis=0)
    x0 = jnp.dot(w0_ref[...], f)
    x0_ref[0] = x0

    @pl.when(jnp.logical_and(b == 0, t == 0))
    def _():
        s0_ref[...] = jnp.zeros_like(s0_ref)
        ss0_ref[...] = jnp.zeros_like(ss0_ref)

    s0_ref[...] += jnp.sum(x0, axis=1, keepdims=True)
    ss0_ref[...] += jnp.sum(x0 * x0, axis=1, keepdims=True)


def _mid_kernel(x0_ref, s0_ref, ss0_ref, g0_ref, b0_ref, w1_ref,
                x1_ref, s1_ref, ss1_ref):
    b = pl.program_id(0)
    mean = s0_ref[...] / NTOT
    var = ss0_ref[...] / NTOT - mean * mean
    rstd = jax.lax.rsqrt(var + 1e-5)
    scale = g0_ref[...] * rstd
    shift = b0_ref[...] - mean * scale
    h = jnp.maximum(x0_ref[0] * scale + shift, 0.0)
    x1 = jnp.dot(w1_ref[...], h)
    x1_ref[0] = x1

    @pl.when(b == 0)
    def _():
        s1_ref[...] = jnp.zeros_like(s1_ref)
        ss1_ref[...] = jnp.zeros_like(ss1_ref)

    s1_ref[...] += jnp.sum(x1, axis=1, keepdims=True)
    ss1_ref[...] += jnp.sum(x1 * x1, axis=1, keepdims=True)


def _out_kernel(x1_ref, s1_ref, ss1_ref, g1_ref, b1_ref, out_ref):
    mean = s1_ref[...] / NTOT
    var = ss1_ref[...] / NTOT - mean * mean
    rstd = jax.lax.rsqrt(var + 1e-5)
    scale = g1_ref[...] * rstd
    shift = b1_ref[...] - mean * scale
    out_ref[0] = jnp.maximum(x1_ref[0] * scale + shift, 0.0)


def kernel(unknown, known, unknow_feats, known_feats, vdfeatures,
           lab_w1, lab_w2, conv0_w, bn0_gamma, bn0_beta,
           conv1_w, bn1_gamma, bn1_beta):
    f32 = jnp.float32
    uT = jnp.transpose(unknown, (0, 2, 1))
    table = jnp.transpose(known_feats, (0, 2, 1)).reshape(B * M, C2)

    gate = pl.pallas_call(
        _gate_kernel,
        grid=(B,),
        in_specs=[
            pl.BlockSpec((1, C1, N), lambda b: (b, 0, 0)),
            pl.BlockSpec((C1 // 4, C1), lambda b: (0, 0)),
            pl.BlockSpec((C1, C1 // 4), lambda b: (0, 0)),
        ],
        out_specs=pl.BlockSpec((1, C1, 1), lambda b: (b, 0, 0)),
        out_shape=jax.ShapeDtypeStruct((B, C1, 1), f32),
    )(unknow_feats, lab_w1, lab_w2)

    gidx, wn = pl.pallas_call(
        _nn_kernel,
        grid=(B, GT),
        in_specs=[
            pl.BlockSpec((1, 3, NT), lambda b, t: (b, 0, t)),
            pl.BlockSpec((1, M, 3), lambda b, t: (b, 0, 0)),
        ],
        out_specs=[
            pl.BlockSpec((1, 3, NT), lambda b, t: (b, 0, t)),
            pl.BlockSpec((1, NT, 3), lambda b, t: (b, t, 0)),
        ],
        out_shape=[
            jax.ShapeDtypeStruct((B, 3, N), jnp.int32),
            jax.ShapeDtypeStruct((B, N, 3), f32),
        ],
    )(uT, known)

    sc_gather = functools.partial(
        pl.kernel,
        mesh=plsc.VectorSubcoreMesh(core_axis_name="c", subcore_axis_name="s"),
        out_type=jax.ShapeDtypeStruct((3 * B * N, C2), f32),
        scratch_types=[
            pltpu.VMEM((QW,), jnp.int32),
            pltpu.VMEM((QW,), jnp.int32),
            pltpu.VMEM((QW,), jnp.int32),
            pltpu.VMEM((CH, C2), f32),
            pltpu.VMEM((CH, C2), f32),
            pltpu.VMEM((CH, C2), f32),
            pltpu.SemaphoreType.DMA,
        ],
    )(_sc_gather_body)
    g3 = sc_gather(table, gidx.reshape(B * 3 * N))

    x0, s0, ss0 = pl.pallas_call(
        _conv0_kernel,
        grid=(B, GT),
        in_specs=[
            pl.BlockSpec((NT, C2), lambda b, t: (0 * (B * GT) + b * GT + t, 0)),
            pl.BlockSpec((NT, C2), lambda b, t: (1 * (B * GT) + b * GT + t, 0)),
            pl.BlockSpec((NT, C2), lambda b, t: (2 * (B * GT) + b * GT + t, 0)),
            pl.BlockSpec((1, NT, 3), lambda b, t: (b, t, 0)),
            pl.BlockSpec((1, C1, NT), lambda b, t: (b, 0, t)),
            pl.BlockSpec((1, VD, NVD), lambda b, t: (b, 0, 0)),
            pl.BlockSpec((1, C1, 1), lambda b, t: (b, 0, 0)),
            pl.BlockSpec((CMID, CIN), lambda b, t: (0, 0)),
        ],
        out_specs=[
            pl.BlockSpec((1, CMID, NT), lambda b, t: (b, 0, t)),
            pl.BlockSpec((CMID, 1), lambda b, t: (0, 0)),
            pl.BlockSpec((CMID, 1), lambda b, t: (0, 0)),
        ],
        out_shape=[
            jax.ShapeDtypeStruct((B, CMID, N), f32),
            jax.ShapeDtypeStruct((CMID, 1), f32),
            jax.ShapeDtypeStruct((CMID, 1), f32),
        ],
    )(g3, g3, g3, wn, unknow_feats, vdfeatures, gate, conv0_w)

    x1, s1, ss1 = pl.pallas_call(
        _mid_kernel,
        grid=(B,),
        in_specs=[
            pl.BlockSpec((1, CMID, N), lambda b: (b, 0, 0)),
            pl.BlockSpec((CMID, 1), lambda b: (0, 0)),
            pl.BlockSpec((CMID, 1), lambda b: (0, 0)),
            pl.BlockSpec((CMID, 1), lambda b: (0, 0)),
            pl.BlockSpec((CMID, 1), lambda b: (0, 0)),
            pl.BlockSpec((COUT, CMID), lambda b: (0, 0)),
        ],
        out_specs=[
            pl.BlockSpec((1, COUT, N), lambda b: (b, 0, 0)),
            pl.BlockSpec((COUT, 1), lambda b: (0, 0)),
            pl.BlockSpec((COUT, 1), lambda b: (0, 0)),
        ],
        out_shape=[
            jax.ShapeDtypeStruct((B, COUT, N), f32),
            jax.ShapeDtypeStruct((COUT, 1), f32),
            jax.ShapeDtypeStruct((COUT, 1), f32),
        ],
    )(x0, s0, ss0, bn0_gamma.reshape(CMID, 1), bn0_beta.reshape(CMID, 1),
      conv1_w)

    out = pl.pallas_call(
        _out_kernel,
        grid=(B,),
        in_specs=[
            pl.BlockSpec((1, COUT, N), lambda b: (b, 0, 0)),
            pl.BlockSpec((COUT, 1), lambda b: (0, 0)),
            pl.BlockSpec((COUT, 1), lambda b: (0, 0)),
            pl.BlockSpec((COUT, 1), lambda b: (0, 0)),
            pl.BlockSpec((COUT, 1), lambda b: (0, 0)),
        ],
        out_specs=pl.BlockSpec((1, COUT, N), lambda b: (b, 0, 0)),
        out_shape=jax.ShapeDtypeStruct((B, COUT, N), f32),
    )(x1, s1, ss1, bn1_gamma.reshape(COUT, 1), bn1_beta.reshape(COUT, 1))

    return out
